# baseline (speedup 1.0000x reference)
import jax
import jax.numpy as jnp
from jax import lax
from jax.experimental import pallas as pl
from jax.experimental.pallas import tpu as pltpu
from jax.experimental.pallas import tpu_sc as plsc

B = 16384
F_CAT = 26
NUM_CAT = 100000
TAB_ROWS = NUM_CAT + 1
F_CONT = 13
DIM = 64
F_TOT = F_CAT + F_CONT

NC, NS, L = 2, 16, 16
NW = NC * NS
BPW = B // NW

CB_A = 64
N_CHUNK_A = BPW // CB_A
RPC = CB_A * F_CAT
G = 128
NG = RPC // G

CB_C = 128
N_CHUNK_C = BPW // CB_C
RCC = CB_C * F_CONT


def _tokenizer_body(xcat_hbm, xc_hbm, tab_hbm, emb_hbm,
                    osrc_hbm, odcat_hbm, odcont_hbm,
                    out_hbm,
                    rows_v, src_idx, dst_idx, xcat_v, xc_v,
                    osrc_v, odcat_v, odcont_v, emb_v, sem):
    cid = lax.axis_index("c")
    sid = lax.axis_index("s")
    wid = sid * NC + cid
    base_b = wid * BPW

    pltpu.sync_copy(osrc_hbm, osrc_v)
    pltpu.sync_copy(odcat_hbm, odcat_v)
    pltpu.sync_copy(odcont_hbm, odcont_v)
    pltpu.sync_copy(emb_hbm, emb_v)

    def cat_chunk(it, carry):
        b0 = base_b + it * CB_A
        pltpu.sync_copy(xcat_hbm.at[pl.ds(b0 * F_CAT, RPC)], xcat_v)
        d_off = b0 * F_TOT

        def idx_grp(k, c2):
            for u in range(G // L):
                s = k * G + u * L
                xi = xcat_v[pl.ds(s, L)]
                src_idx[k, pl.ds(u * L, L)] = xi + osrc_v[pl.ds(s, L)]
                dst_idx[k, pl.ds(u * L, L)] = odcat_v[pl.ds(s, L)] + d_off
            return c2

        lax.fori_loop(0, NG, idx_grp, 0)

        gathers = [
            pltpu.async_copy(tab_hbm.at[src_idx.at[k]],
                             rows_v.at[pl.ds(k * G, G)], sem)
            for k in range(NG)
        ]
        for h in gathers:
            h.wait()
        scatters = [
            pltpu.async_copy(rows_v.at[pl.ds(k * G, G)],
                             out_hbm.at[dst_idx.at[k]], sem)
            for k in range(NG)
        ]
        for h in scatters:
            h.wait()
        return carry

    lax.fori_loop(0, N_CHUNK_A, cat_chunk, 0)

    def cont_chunk(it, carry):
        b0 = base_b + it * CB_C
        pltpu.sync_copy(xc_hbm.at[pl.ds(b0 * F_CONT, RCC)], xc_v)
        d_off = b0 * F_TOT

        def grp_body(g, c2):
            s = g * L
            v16 = xc_v[pl.ds(s, L)]
            for lane in range(L):
                r = s + lane
                f = lax.rem(r, F_CONT)
                spl = jnp.full((L,), v16[lane], jnp.float32)
                for q in range(DIM // L):
                    rows_v[r, pl.ds(q * L, L)] = (
                        spl * emb_v[pl.ds(f * DIM + q * L, L)])
            return c2

        lax.fori_loop(0, RCC // L, grp_body, 0)

        def idx_grp(k, c2):
            for u in range(G // L):
                s = k * G + u * L
                dst_idx[k, pl.ds(u * L, L)] = odcont_v[pl.ds(s, L)] + d_off
            return c2

        lax.fori_loop(0, NG, idx_grp, 0)

        scatters = [
            pltpu.async_copy(rows_v.at[pl.ds(k * G, G)],
                             out_hbm.at[dst_idx.at[k]], sem)
            for k in range(NG)
        ]
        for h in scatters:
            h.wait()
        return carry

    lax.fori_loop(0, N_CHUNK_C, cont_chunk, 0)


_mesh = plsc.VectorSubcoreMesh(core_axis_name="c", subcore_axis_name="s",
                               num_cores=NC, num_subcores=NS)

_sc_call = pl.kernel(
    _tokenizer_body,
    out_type=jax.ShapeDtypeStruct((B * F_TOT, DIM), jnp.float32),
    mesh=_mesh,
    compiler_params=pltpu.CompilerParams(use_tc_tiling_on_sc=False),
    scratch_types=[
        pltpu.VMEM((RPC, DIM), jnp.float32),
        pltpu.VMEM((NG, G), jnp.int32),
        pltpu.VMEM((NG, G), jnp.int32),
        pltpu.VMEM((RPC,), jnp.int32),
        pltpu.VMEM((RCC,), jnp.float32),
        pltpu.VMEM((RPC,), jnp.int32),
        pltpu.VMEM((RPC,), jnp.int32),
        pltpu.VMEM((RCC,), jnp.int32),
        pltpu.VMEM((F_CONT * DIM,), jnp.float32),
        pltpu.SemaphoreType.DMA,
    ],
)


def kernel(x_categ, x_cont, cat_tables, cont_embeds):
    tab = cat_tables.reshape(F_CAT * TAB_ROWS, DIM)
    xcat = x_categ.reshape(B * F_CAT)
    xc = x_cont.reshape(B * F_CONT)
    emb = cont_embeds.reshape(F_CONT * DIM)
    j = jnp.arange(RPC, dtype=jnp.int32)
    osrc = (j % F_CAT) * TAB_ROWS
    odcat = (j // F_CAT) * F_TOT + (j % F_CAT)
    odcont = (j // F_CONT) * F_TOT + F_CAT + (j % F_CONT)
    out = _sc_call(xcat, xc, tab, emb, osrc, odcat, odcont)
    return out.reshape(B, F_TOT, DIM)
